# baseline (speedup 1.0000x reference)
import functools

import jax
import jax.numpy as jnp
from jax import lax
from jax.experimental import pallas as pl
from jax.experimental.pallas import tpu as pltpu
from jax.experimental.pallas import tpu_sc as plsc

BATCH = 16384
EMB_DIM = 32
LANES = 16

_INFO = plsc.get_sparse_core_info()
_NC = _INFO.num_cores
_NS = _INFO.num_subcores
NW = _NC * _NS
BPW = BATCH // NW
CW = 128
CH = BPW // CW
GROUPS = BPW // LANES


def _sc_body(u_idx_hbm, p_idx_hbm, n_idx_hbm, uemb_hbm, iemb_hbm, bias_hbm,
             pos_out, neg_out,
             uidx_v, pidx_v, nidx_v, ue_v, pe_v, ne_v, pb_v, nb_v,
             ps_v, ns_v, sem):
    wid = lax.axis_index("s") * _NC + lax.axis_index("c")
    base = wid * BPW

    pltpu.sync_copy(u_idx_hbm.at[wid], uidx_v)
    pltpu.sync_copy(p_idx_hbm.at[wid], pidx_v)
    pltpu.sync_copy(n_idx_hbm.at[wid], nidx_v)

    copies = []
    for c in range(CH):
        sl = pl.ds(c * CW, CW)
        copies.append(pltpu.async_copy(uemb_hbm.at[uidx_v.at[c]], ue_v.at[sl], sem))
        copies.append(pltpu.async_copy(iemb_hbm.at[pidx_v.at[c]], pe_v.at[sl], sem))
        copies.append(pltpu.async_copy(iemb_hbm.at[nidx_v.at[c]], ne_v.at[sl], sem))
        copies.append(pltpu.async_copy(bias_hbm.at[pidx_v.at[c]], pb_v.at[sl], sem))
        copies.append(pltpu.async_copy(bias_hbm.at[nidx_v.at[c]], nb_v.at[sl], sem))
    for cp in copies:
        cp.wait()

    iot = lax.iota(jnp.int32, LANES)

    def grp(g, carry):
        rb = g * LANES
        rows = rb + iot
        accp = pb_v[pl.ds(rb, LANES)]
        accn = nb_v[pl.ds(rb, LANES)]
        for d in range(EMB_DIM):
            dsplat = jnp.full((LANES,), d, jnp.int32)
            uv = plsc.load_gather(ue_v, [rows, dsplat])
            pv = plsc.load_gather(pe_v, [rows, dsplat])
            nv = plsc.load_gather(ne_v, [rows, dsplat])
            accp = accp + uv * pv
            accn = accn + uv * nv
        ps_v[pl.ds(rb, LANES)] = accp
        ns_v[pl.ds(rb, LANES)] = accn
        return carry

    lax.fori_loop(0, GROUPS, grp, 0)

    pltpu.sync_copy(ps_v, pos_out.at[pl.ds(base, BPW)])
    pltpu.sync_copy(ns_v, neg_out.at[pl.ds(base, BPW)])


@jax.jit
def _bprmf_sc(u_idx, p_idx, n_idx, uemb, iemb, bias_flat):
    mesh = plsc.VectorSubcoreMesh(core_axis_name="c", subcore_axis_name="s")
    f = functools.partial(
        pl.kernel,
        mesh=mesh,
        out_type=(
            jax.ShapeDtypeStruct((BATCH,), jnp.float32),
            jax.ShapeDtypeStruct((BATCH,), jnp.float32),
        ),
        scratch_types=[
            pltpu.VMEM((CH, CW), jnp.int32),
            pltpu.VMEM((CH, CW), jnp.int32),
            pltpu.VMEM((CH, CW), jnp.int32),
            pltpu.VMEM((BPW, EMB_DIM), jnp.float32),
            pltpu.VMEM((BPW, EMB_DIM), jnp.float32),
            pltpu.VMEM((BPW, EMB_DIM), jnp.float32),
            pltpu.VMEM((BPW,), jnp.float32),
            pltpu.VMEM((BPW,), jnp.float32),
            pltpu.VMEM((BPW,), jnp.float32),
            pltpu.VMEM((BPW,), jnp.float32),
            pltpu.SemaphoreType.DMA,
        ],
        compiler_params=pltpu.CompilerParams(
            use_tc_tiling_on_sc=False,
            needs_layout_passes=False,
        ),
    )(_sc_body)
    return f(u_idx, p_idx, n_idx, uemb, iemb, bias_flat)


def kernel(users, pos_items, neg_items, user_embedding, item_embedding, item_bias):
    u_idx = users.astype(jnp.int32).reshape(NW, CH, CW)
    p_idx = pos_items.astype(jnp.int32).reshape(NW, CH, CW)
    n_idx = neg_items.astype(jnp.int32).reshape(NW, CH, CW)
    bias_flat = item_bias.reshape(-1)
    return _bprmf_sc(u_idx, p_idx, n_idx, user_embedding, item_embedding, bias_flat)


# speedup vs baseline: 1.0693x; 1.0693x over previous
import functools

import jax
import jax.numpy as jnp
from jax import lax
from jax.experimental import pallas as pl
from jax.experimental.pallas import tpu as pltpu
from jax.experimental.pallas import tpu_sc as plsc

BATCH = 16384
EMB_DIM = 32
LANES = 16

_INFO = plsc.get_sparse_core_info()
_NC = _INFO.num_cores
_NS = _INFO.num_subcores
NW = _NC * _NS
BPW = BATCH // NW
HALF = BPW // 2
HGROUPS = HALF // LANES


def _sc_body(u_idx_hbm, p_idx_hbm, n_idx_hbm, uemb_hbm, iemb_hbm, bias_hbm,
             pos_out, neg_out,
             idx_v, ue_v, it_v, ib_v, ps_v, ns_v, sem):
    wid = lax.axis_index("s") * _NC + lax.axis_index("c")
    base = wid * BPW
    iot = lax.iota(jnp.int32, LANES)
    zeros16 = jnp.zeros((LANES,), jnp.int32)

    def stage_idx(idx_hbm, off):
        pltpu.sync_copy(idx_hbm.at[pl.ds(base + off, HALF)], idx_v)

    def fire_user(g, carry):
        v = idx_v[pl.ds(g * LANES, LANES)]
        for j in range(LANES):
            r = v[j]
            i = g * LANES + j
            pltpu.async_copy(uemb_hbm.at[pl.ds(r, 1)], ue_v.at[pl.ds(i, 1)], sem)
        return carry

    def fire_item(g, carry):
        v = idx_v[pl.ds(g * LANES, LANES)]
        for j in range(LANES):
            r = v[j]
            i = g * LANES + j
            pltpu.async_copy(iemb_hbm.at[pl.ds(r, 1)], it_v.at[pl.ds(i, 1)], sem)
            pltpu.async_copy(bias_hbm.at[pl.ds(r, 1)], ib_v.at[pl.ds(i, 1)], sem)
        return carry

    def compute(off, sc_v):
        def grp(g, carry):
            rows = g * LANES + iot
            acc = plsc.load_gather(ib_v, [rows, zeros16])
            for d in range(EMB_DIM):
                dsplat = jnp.full((LANES,), d, jnp.int32)
                uv = plsc.load_gather(ue_v, [rows, dsplat])
                iv = plsc.load_gather(it_v, [rows, dsplat])
                acc = acc + uv * iv
            sc_v[pl.ds(off + g * LANES, LANES)] = acc
            return carry

        lax.fori_loop(0, HGROUPS, grp, 0)

    for uh in range(2):
        off = uh * HALF
        stage_idx(u_idx_hbm, off)
        lax.fori_loop(0, HGROUPS, fire_user, 0)
        pltpu.make_async_copy(uemb_hbm.at[pl.ds(0, HALF)], ue_v, sem).wait()
        for idx_hbm, sc_v in ((p_idx_hbm, ps_v), (n_idx_hbm, ns_v)):
            stage_idx(idx_hbm, off)
            lax.fori_loop(0, HGROUPS, fire_item, 0)
            pltpu.make_async_copy(iemb_hbm.at[pl.ds(0, HALF)], it_v, sem).wait()
            pltpu.make_async_copy(bias_hbm.at[pl.ds(0, HALF)], ib_v, sem).wait()
            compute(off, sc_v)

    pltpu.sync_copy(ps_v, pos_out.at[pl.ds(base, BPW)])
    pltpu.sync_copy(ns_v, neg_out.at[pl.ds(base, BPW)])


@jax.jit
def _bprmf_sc(u_idx, p_idx, n_idx, uemb, iemb, bias):
    mesh = plsc.VectorSubcoreMesh(core_axis_name="c", subcore_axis_name="s")
    f = functools.partial(
        pl.kernel,
        mesh=mesh,
        out_type=(
            jax.ShapeDtypeStruct((BATCH,), jnp.float32),
            jax.ShapeDtypeStruct((BATCH,), jnp.float32),
        ),
        scratch_types=[
            pltpu.VMEM((HALF,), jnp.int32),
            pltpu.VMEM((HALF, EMB_DIM), jnp.float32),
            pltpu.VMEM((HALF, EMB_DIM), jnp.float32),
            pltpu.VMEM((HALF, 1), jnp.float32),
            pltpu.VMEM((BPW,), jnp.float32),
            pltpu.VMEM((BPW,), jnp.float32),
            pltpu.SemaphoreType.DMA,
        ],
        compiler_params=pltpu.CompilerParams(
            needs_layout_passes=False,
        ),
    )(_sc_body)
    return f(u_idx, p_idx, n_idx, uemb, iemb, bias)


def kernel(users, pos_items, neg_items, user_embedding, item_embedding, item_bias):
    return _bprmf_sc(
        users.astype(jnp.int32),
        pos_items.astype(jnp.int32),
        neg_items.astype(jnp.int32),
        user_embedding,
        item_embedding,
        item_bias,
    )


# speedup vs baseline: 1.3500x; 1.2624x over previous
import functools

import jax
import jax.numpy as jnp
from jax import lax
from jax.experimental import pallas as pl
from jax.experimental.pallas import tpu as pltpu
from jax.experimental.pallas import tpu_sc as plsc

BATCH = 16384
EMB_DIM = 32
LANES = 16

_INFO = plsc.get_sparse_core_info()
_NC = _INFO.num_cores
_NS = _INFO.num_subcores
NW = _NC * _NS
BPW = BATCH // NW
HALF = BPW // 2
HGROUPS = HALF // LANES
CW = 128


def _sc_body(u_idx_hbm, p_idx_hbm, n_idx_hbm, uemb_hbm, iemb_hbm, bias_hbm,
             pos_out, neg_out,
             idx_v, ue_v, it_v, pb_v, ps_v, ns_v, sem):
    wid = lax.axis_index("s") * _NC + lax.axis_index("c")
    base = wid * BPW
    iot = lax.iota(jnp.int32, LANES)
    zeros16 = jnp.zeros((LANES,), jnp.int32)

    def stage_idx(idx_hbm, off):
        pltpu.sync_copy(idx_hbm.at[pl.ds(base + off, HALF)], idx_v)

    def fire_user(g, carry):
        v = idx_v[pl.ds(g * LANES, LANES)]
        for j in range(LANES):
            r = v[j]
            i = g * LANES + j
            pltpu.async_copy(uemb_hbm.at[pl.ds(r, 1)], ue_v.at[pl.ds(i, 1)], sem)
        return carry

    def fire_item(g, carry):
        v = idx_v[pl.ds(g * LANES, LANES)]
        for j in range(LANES):
            r = v[j]
            i = g * LANES + j
            pltpu.async_copy(iemb_hbm.at[pl.ds(r, 1)], it_v.at[pl.ds(i, 1)], sem)
        return carry

    def compute(off, sc_v):
        def grp(g, carry):
            rows = g * LANES + iot
            acc = pb_v[pl.ds(g * LANES, LANES)]
            for d in range(EMB_DIM):
                dsplat = jnp.full((LANES,), d, jnp.int32)
                uv = plsc.load_gather(ue_v, [rows, dsplat])
                iv = plsc.load_gather(it_v, [rows, dsplat])
                acc = acc + uv * iv
            sc_v[pl.ds(off + g * LANES, LANES)] = acc
            return carry

        lax.fori_loop(0, HGROUPS, grp, 0)

    for uh in range(2):
        off = uh * HALF
        stage_idx(u_idx_hbm, off)
        lax.fori_loop(0, HGROUPS, fire_user, 0)
        pltpu.make_async_copy(uemb_hbm.at[pl.ds(0, HALF)], ue_v, sem).wait()
        for idx_hbm, sc_v in ((p_idx_hbm, ps_v), (n_idx_hbm, ns_v)):
            stage_idx(idx_hbm, off)
            lax.fori_loop(0, HGROUPS, fire_item, 0)
            for c in range(HALF // CW):
                sl = pl.ds(c * CW, CW)
                pltpu.async_copy(bias_hbm.at[idx_v.at[sl]], pb_v.at[sl], sem)
            pltpu.make_async_copy(iemb_hbm.at[pl.ds(0, HALF)], it_v, sem).wait()
            pltpu.make_async_copy(bias_hbm.at[pl.ds(0, HALF)], pb_v, sem).wait()
            compute(off, sc_v)

    pltpu.sync_copy(ps_v, pos_out.at[pl.ds(base, BPW)])
    pltpu.sync_copy(ns_v, neg_out.at[pl.ds(base, BPW)])


@jax.jit
def _bprmf_sc(u_idx, p_idx, n_idx, uemb, iemb, bias1):
    mesh = plsc.VectorSubcoreMesh(core_axis_name="c", subcore_axis_name="s")
    f = functools.partial(
        pl.kernel,
        mesh=mesh,
        out_type=(
            jax.ShapeDtypeStruct((BATCH,), jnp.float32),
            jax.ShapeDtypeStruct((BATCH,), jnp.float32),
        ),
        scratch_types=[
            pltpu.VMEM((HALF,), jnp.int32),
            pltpu.VMEM((HALF, EMB_DIM), jnp.float32),
            pltpu.VMEM((HALF, EMB_DIM), jnp.float32),
            pltpu.VMEM((HALF,), jnp.float32),
            pltpu.VMEM((BPW,), jnp.float32),
            pltpu.VMEM((BPW,), jnp.float32),
            pltpu.SemaphoreType.DMA,
        ],
        compiler_params=pltpu.CompilerParams(
            needs_layout_passes=False,
        ),
    )(_sc_body)
    return f(u_idx, p_idx, n_idx, uemb, iemb, bias1)


def kernel(users, pos_items, neg_items, user_embedding, item_embedding, item_bias):
    return _bprmf_sc(
        users.astype(jnp.int32),
        pos_items.astype(jnp.int32),
        neg_items.astype(jnp.int32),
        user_embedding,
        item_embedding,
        item_bias.reshape(-1),
    )
